# baseline (speedup 1.0000x reference)
import functools

import numpy as np

import jax
import jax.numpy as jnp
from jax.experimental import pallas as pl
from jax.experimental.pallas import tpu as pltpu

TWO_PI = 2.0 * np.pi


def _aug_kernel(x_ref, row_ref, col_ref, *, neg2c, c):
    x = x_ref[...]
    sa = jnp.sum(x * x, axis=1, keepdims=True)
    nrm = c * sa + 0.5
    row_ref[:, 0:64] = x * neg2c
    row_ref[:, 64:65] = nrm
    row_ref[:, 65:66] = jnp.ones_like(nrm)
    col_ref[:, 0:64] = x
    col_ref[:, 64:65] = jnp.ones_like(nrm)
    col_ref[:, 65:66] = nrm


def _main_kernel(row_ref, col_ref, means_ref, var_ref, lp_ref, out_ref,
                 acc_ref, *, ni, jj_n, c, scale_a, gamma, n, d, k):
    i = pl.program_id(0)
    jj = pl.program_id(1)

    @pl.when(jj == 0)
    def _init():
        inv_c = 1.0 / c
        x = row_ref[:, 0:64] * (-0.5 * inv_c)
        sa = (row_ref[:, 64:65] - 0.5) * inv_c
        means = means_ref[...]
        var_row = var_ref[...]
        lp_row = lp_ref[...]

        m_gram = jax.lax.dot_general(means, means, (((1,), (1,)), ((), ())),
                                     preferred_element_type=jnp.float32)
        eye = (jax.lax.broadcasted_iota(jnp.int32, (k, k), 0)
               == jax.lax.broadcasted_iota(jnp.int32, (k, k), 1)
               ).astype(jnp.float32)
        sm_col = jnp.sum(m_gram * eye, axis=1, keepdims=True)
        sm_row = jnp.sum(m_gram * eye, axis=0, keepdims=True)

        e = jnp.exp(lp_row - jnp.max(lp_row))
        p_row = e / jnp.sum(e)

        gc = jax.lax.dot_general(x, means, (((1,), (1,)), ((), ())),
                                 preferred_element_type=jnp.float32)
        d2c = jnp.maximum(sa + sm_row - 2.0 * gc, 0.0)
        den_c = 2.0 * (var_row + 2.0 * gamma)
        phi_c = jax.lax.rsqrt(1.0 + d2c * ((4.0 / (2 * d - 3)) / den_c))
        coef_c = (2.0 / n) * p_row * jax.lax.rsqrt(
            TWO_PI * (var_row + 2.0 * gamma))
        c_scalar = jnp.sum(coef_c * phi_c)

        var_col = jnp.sum(var_row * eye, axis=1, keepdims=True)
        p_col = jnp.sum(p_row * eye, axis=1, keepdims=True)
        var_mat = var_col + var_row
        b1 = jnp.maximum(sm_col + sm_row - 2.0 * m_gram, 0.0)
        b2 = jax.lax.rsqrt(1.0 + b1 * (4.0 / (2 * d - 3))
                           / (2.0 * var_mat + 4.0 * gamma))
        b3 = p_col * p_row * jax.lax.rsqrt(TWO_PI * (var_mat + 2.0 * gamma))
        b_scalar = jnp.sum(b3 * b2)

        first_row = jnp.where(i == 0, 1.0, 0.0)
        init_val = (first_row * b_scalar - c_scalar) * (1.0 / 1024.0)
        acc_ref[...] = jnp.full((8, 128), init_val, dtype=jnp.float32)

    arg = jax.lax.dot_general(row_ref[...], col_ref[...],
                              (((1,), (1,)), ((), ())),
                              preferred_element_type=jnp.float32)
    phi = jax.lax.rsqrt(jnp.maximum(arg, 1.0))
    ti = phi.shape[0]
    red = phi.reshape(ti // 8, 8, phi.shape[1] // 128, 128).sum(axis=(0, 2))
    edge = jnp.logical_or(jj == 0, jj == jj_n - 1)
    mult = jnp.where(edge, scale_a, 2.0 * scale_a)
    acc_ref[...] += mult * red

    @pl.when(jj == jj_n - 1)
    def _flush():
        out_ref[...] = jnp.sum(acc_ref[...], axis=0, keepdims=True
                               ).reshape(1, 1, 128)


@jax.jit
def kernel(X, means, variances, logit_probs):
    n, d = X.shape
    k = means.shape[0]
    gamma = float(np.power(4.0 / (3.0 * n / k), 0.4))
    c = 1.0 / ((2 * d - 3) * gamma)
    scale_a = 1.0 / (n * n * np.sqrt(TWO_PI * 2.0 * gamma))

    ti = 512
    ni = n // ti
    jj_n = ni // 2 + 1

    tb = 2048
    aug_row, aug_col = pl.pallas_call(
        functools.partial(_aug_kernel, neg2c=-2.0 * c, c=c),
        grid=(n // tb,),
        in_specs=[pl.BlockSpec((tb, d), lambda b: (b, 0))],
        out_specs=[pl.BlockSpec((tb, 66), lambda b: (b, 0)),
                   pl.BlockSpec((tb, 66), lambda b: (b, 0))],
        out_shape=[jax.ShapeDtypeStruct((n, 66), jnp.float32),
                   jax.ShapeDtypeStruct((n, 66), jnp.float32)],
        compiler_params=pltpu.CompilerParams(
            dimension_semantics=("parallel",)),
        name="cs_aug",
    )(X)

    partials = pl.pallas_call(
        functools.partial(_main_kernel, ni=ni, jj_n=jj_n, c=c,
                          scale_a=scale_a, gamma=gamma, n=n, d=d, k=k),
        grid=(ni, jj_n),
        in_specs=[
            pl.BlockSpec((ti, 66), lambda i, jj: (i, 0)),
            pl.BlockSpec((ti, 66), lambda i, jj: ((i + jj) % ni, 0)),
            pl.BlockSpec((k, d), lambda i, jj: (0, 0)),
            pl.BlockSpec((1, k), lambda i, jj: (0, 0)),
            pl.BlockSpec((1, k), lambda i, jj: (0, 0)),
        ],
        out_specs=pl.BlockSpec((1, 1, 128), lambda i, jj: (i, 0, 0)),
        out_shape=jax.ShapeDtypeStruct((ni, 1, 128), jnp.float32),
        scratch_shapes=[pltpu.VMEM((8, 128), jnp.float32)],
        compiler_params=pltpu.CompilerParams(
            dimension_semantics=("parallel", "arbitrary")),
        name="cs_pairwise",
    )(aug_row, aug_col, means,
      variances.reshape(1, k), logit_probs.reshape(1, k))

    return jnp.log(jnp.sum(partials))


# speedup vs baseline: 1.1843x; 1.1843x over previous
import functools

import numpy as np

import jax
import jax.numpy as jnp
from jax.experimental import pallas as pl
from jax.experimental.pallas import tpu as pltpu

TWO_PI = 2.0 * np.pi


def _aug_kernel(x_ref, row_ref, col_ref, *, neg2c, c):
    x = x_ref[...]
    sa = jnp.sum(x * x, axis=1, keepdims=True)
    nrm = c * sa + 0.5
    row_ref[:, 0:64] = x * neg2c
    row_ref[:, 64:65] = nrm
    row_ref[:, 65:66] = jnp.ones_like(nrm)
    col_ref[:, 0:64] = x
    col_ref[:, 64:65] = jnp.ones_like(nrm)
    col_ref[:, 65:66] = nrm


def _main_kernel(row_ref, col_ref, means_ref, var_ref, lp_ref, out_ref,
                 acc_ref, *, ni, jj_n, c, scale_a, gamma, n, d, k):
    i = pl.program_id(0)
    jj = pl.program_id(1)

    @pl.when(jj == 0)
    def _init():
        inv_c = 1.0 / c
        x = row_ref[:, 0:64] * (-0.5 * inv_c)
        sa = (row_ref[:, 64:65] - 0.5) * inv_c
        means = means_ref[...]
        var_row = var_ref[...]
        lp_row = lp_ref[...]

        m_gram = jax.lax.dot_general(means, means, (((1,), (1,)), ((), ())),
                                     preferred_element_type=jnp.float32)
        eye = (jax.lax.broadcasted_iota(jnp.int32, (k, k), 0)
               == jax.lax.broadcasted_iota(jnp.int32, (k, k), 1)
               ).astype(jnp.float32)
        sm_col = jnp.sum(m_gram * eye, axis=1, keepdims=True)
        sm_row = jnp.sum(m_gram * eye, axis=0, keepdims=True)

        e = jnp.exp(lp_row - jnp.max(lp_row))
        p_row = e / jnp.sum(e)

        gc = jax.lax.dot_general(x, means, (((1,), (1,)), ((), ())),
                                 preferred_element_type=jnp.float32)
        d2c = jnp.maximum(sa + sm_row - 2.0 * gc, 0.0)
        den_c = 2.0 * (var_row + 2.0 * gamma)
        phi_c = jax.lax.rsqrt(1.0 + d2c * ((4.0 / (2 * d - 3)) / den_c))
        coef_c = (2.0 / n) * p_row * jax.lax.rsqrt(
            TWO_PI * (var_row + 2.0 * gamma))
        c_scalar = jnp.sum(coef_c * phi_c)

        var_col = jnp.sum(var_row * eye, axis=1, keepdims=True)
        p_col = jnp.sum(p_row * eye, axis=1, keepdims=True)
        var_mat = var_col + var_row
        b1 = jnp.maximum(sm_col + sm_row - 2.0 * m_gram, 0.0)
        b2 = jax.lax.rsqrt(1.0 + b1 * (4.0 / (2 * d - 3))
                           / (2.0 * var_mat + 4.0 * gamma))
        b3 = p_col * p_row * jax.lax.rsqrt(TWO_PI * (var_mat + 2.0 * gamma))
        b_scalar = jnp.sum(b3 * b2)

        first_row = jnp.where(i == 0, 1.0, 0.0)
        ti = row_ref.shape[0]
        init_val = (first_row * b_scalar - c_scalar) / (ti * 128.0)
        acc_ref[...] = jnp.full((ti, 128), init_val, dtype=jnp.float32)

    arg = jax.lax.dot_general(row_ref[...], col_ref[...],
                              (((1,), (1,)), ((), ())),
                              preferred_element_type=jnp.float32)
    phi = jax.lax.rsqrt(jnp.maximum(arg, 1.0))
    red = phi[:, 0:128]
    for g in range(128, phi.shape[1], 128):
        red = red + phi[:, g:g + 128]
    edge = jnp.logical_or(jj == 0, jj == jj_n - 1)
    mult = jnp.where(edge, scale_a, 2.0 * scale_a)
    acc_ref[...] += mult * red

    @pl.when(jj == jj_n - 1)
    def _flush():
        out_ref[...] = jnp.sum(acc_ref[...], axis=0, keepdims=True
                               ).reshape(1, 1, 128)


@jax.jit
def kernel(X, means, variances, logit_probs):
    n, d = X.shape
    k = means.shape[0]
    gamma = float(np.power(4.0 / (3.0 * n / k), 0.4))
    c = 1.0 / ((2 * d - 3) * gamma)
    scale_a = 1.0 / (n * n * np.sqrt(TWO_PI * 2.0 * gamma))

    ti = 512
    ni = n // ti
    jj_n = ni // 2 + 1

    tb = 2048
    aug_row, aug_col = pl.pallas_call(
        functools.partial(_aug_kernel, neg2c=-2.0 * c, c=c),
        grid=(n // tb,),
        in_specs=[pl.BlockSpec((tb, d), lambda b: (b, 0))],
        out_specs=[pl.BlockSpec((tb, 66), lambda b: (b, 0)),
                   pl.BlockSpec((tb, 66), lambda b: (b, 0))],
        out_shape=[jax.ShapeDtypeStruct((n, 66), jnp.float32),
                   jax.ShapeDtypeStruct((n, 66), jnp.float32)],
        compiler_params=pltpu.CompilerParams(
            dimension_semantics=("parallel",)),
        name="cs_aug",
    )(X)

    partials = pl.pallas_call(
        functools.partial(_main_kernel, ni=ni, jj_n=jj_n, c=c,
                          scale_a=scale_a, gamma=gamma, n=n, d=d, k=k),
        grid=(ni, jj_n),
        in_specs=[
            pl.BlockSpec((ti, 66), lambda i, jj: (i, 0)),
            pl.BlockSpec((ti, 66), lambda i, jj: ((i + jj) % ni, 0)),
            pl.BlockSpec((k, d), lambda i, jj: (0, 0)),
            pl.BlockSpec((1, k), lambda i, jj: (0, 0)),
            pl.BlockSpec((1, k), lambda i, jj: (0, 0)),
        ],
        out_specs=pl.BlockSpec((1, 1, 128), lambda i, jj: (i, 0, 0)),
        out_shape=jax.ShapeDtypeStruct((ni, 1, 128), jnp.float32),
        scratch_shapes=[pltpu.VMEM((ti, 128), jnp.float32)],
        compiler_params=pltpu.CompilerParams(
            dimension_semantics=("parallel", "arbitrary")),
        name="cs_pairwise",
    )(aug_row, aug_col, means,
      variances.reshape(1, k), logit_probs.reshape(1, k))

    return jnp.log(jnp.sum(partials))


# speedup vs baseline: 1.2235x; 1.0331x over previous
import functools

import numpy as np

import jax
import jax.numpy as jnp
from jax.experimental import pallas as pl
from jax.experimental.pallas import tpu as pltpu

TWO_PI = 2.0 * np.pi


def _aug_kernel(x_ref, means_ref, var_ref, lp_ref, row_ref, col_ref, bc_ref,
                *, c, gamma, n, d, k):
    x = x_ref[...]
    sa = jnp.sum(x * x, axis=1, keepdims=True)
    nrm = c * sa + 0.5
    row_ref[:, 0:64] = x * (-2.0 * c)
    row_ref[:, 64:65] = nrm
    row_ref[:, 65:66] = jnp.ones_like(nrm)
    col_ref[:, 0:64] = x
    col_ref[:, 64:65] = jnp.ones_like(nrm)
    col_ref[:, 65:66] = nrm

    means = means_ref[...]
    var_row = var_ref[...]
    lp_row = lp_ref[...]

    m_gram = jax.lax.dot_general(means, means, (((1,), (1,)), ((), ())),
                                 preferred_element_type=jnp.float32)
    eye = (jax.lax.broadcasted_iota(jnp.int32, (k, k), 0)
           == jax.lax.broadcasted_iota(jnp.int32, (k, k), 1)
           ).astype(jnp.float32)
    sm_col = jnp.sum(m_gram * eye, axis=1, keepdims=True)
    sm_row = jnp.sum(m_gram * eye, axis=0, keepdims=True)

    e = jnp.exp(lp_row - jnp.max(lp_row))
    p_row = e / jnp.sum(e)

    gc = jax.lax.dot_general(x, means, (((1,), (1,)), ((), ())),
                             preferred_element_type=jnp.float32)
    d2c = jnp.maximum(sa + sm_row - 2.0 * gc, 0.0)
    den_c = 2.0 * (var_row + 2.0 * gamma)
    phi_c = jax.lax.rsqrt(1.0 + d2c * ((4.0 / (2 * d - 3)) / den_c))
    coef_c = (2.0 / n) * p_row * jax.lax.rsqrt(
        TWO_PI * (var_row + 2.0 * gamma))
    c_scalar = jnp.sum(coef_c * phi_c)

    var_col = jnp.sum(var_row * eye, axis=1, keepdims=True)
    p_col = jnp.sum(p_row * eye, axis=1, keepdims=True)
    var_mat = var_col + var_row
    b1 = jnp.maximum(sm_col + sm_row - 2.0 * m_gram, 0.0)
    b2 = jax.lax.rsqrt(1.0 + b1 * (4.0 / (2 * d - 3))
                       / (2.0 * var_mat + 4.0 * gamma))
    b3 = p_col * p_row * jax.lax.rsqrt(TWO_PI * (var_mat + 2.0 * gamma))
    b_scalar = jnp.sum(b3 * b2)

    first = jnp.where(pl.program_id(0) == 0, 1.0, 0.0)
    bc_ref[...] = jnp.full((1, 1, 128),
                           (first * b_scalar - c_scalar) / 128.0,
                           dtype=jnp.float32)


def _main_kernel(row_ref, col_ref, out_ref, acc_ref, *, jj_n, scale_a):
    jj = pl.program_id(1)

    @pl.when(jj == 0)
    def _init():
        acc_ref[...] = jnp.zeros_like(acc_ref)

    arg = jax.lax.dot_general(row_ref[...], col_ref[...],
                              (((1,), (1,)), ((), ())),
                              preferred_element_type=jnp.float32)
    phi = jax.lax.rsqrt(jnp.maximum(arg, 1.0))
    red = phi[:, 0:128]
    for g in range(128, phi.shape[1], 128):
        red = red + phi[:, g:g + 128]
    edge = jnp.logical_or(jj == 0, jj == jj_n - 1)
    mult = jnp.where(edge, scale_a, 2.0 * scale_a)
    acc_ref[...] += mult * red

    @pl.when(jj == jj_n - 1)
    def _flush():
        out_ref[...] = jnp.sum(acc_ref[...], axis=0, keepdims=True
                               ).reshape(1, 1, 128)


@jax.jit
def kernel(X, means, variances, logit_probs):
    n, d = X.shape
    k = means.shape[0]
    gamma = float(np.power(4.0 / (3.0 * n / k), 0.4))
    c = 1.0 / ((2 * d - 3) * gamma)
    scale_a = 1.0 / (n * n * np.sqrt(TWO_PI * 2.0 * gamma))

    ti = 512
    ni = n // ti
    jj_n = ni // 2 + 1

    tb = 2048
    nb = n // tb
    aug_row, aug_col, bc_parts = pl.pallas_call(
        functools.partial(_aug_kernel, c=c, gamma=gamma, n=n, d=d, k=k),
        grid=(nb,),
        in_specs=[pl.BlockSpec((tb, d), lambda b: (b, 0)),
                  pl.BlockSpec((k, d), lambda b: (0, 0)),
                  pl.BlockSpec((1, k), lambda b: (0, 0)),
                  pl.BlockSpec((1, k), lambda b: (0, 0))],
        out_specs=[pl.BlockSpec((tb, 66), lambda b: (b, 0)),
                   pl.BlockSpec((tb, 66), lambda b: (b, 0)),
                   pl.BlockSpec((1, 1, 128), lambda b: (b, 0, 0))],
        out_shape=[jax.ShapeDtypeStruct((n, 66), jnp.float32),
                   jax.ShapeDtypeStruct((n, 66), jnp.float32),
                   jax.ShapeDtypeStruct((nb, 1, 128), jnp.float32)],
        compiler_params=pltpu.CompilerParams(
            dimension_semantics=("arbitrary",)),
        name="cs_aug",
    )(X, means, variances.reshape(1, k), logit_probs.reshape(1, k))

    partials = pl.pallas_call(
        functools.partial(_main_kernel, jj_n=jj_n, scale_a=scale_a),
        grid=(ni, jj_n),
        in_specs=[
            pl.BlockSpec((ti, 66), lambda i, jj: (i, 0)),
            pl.BlockSpec((ti, 66), lambda i, jj: ((i + jj) % ni, 0)),
        ],
        out_specs=pl.BlockSpec((1, 1, 128), lambda i, jj: (i, 0, 0)),
        out_shape=jax.ShapeDtypeStruct((ni, 1, 128), jnp.float32),
        scratch_shapes=[pltpu.VMEM((ti, 128), jnp.float32)],
        compiler_params=pltpu.CompilerParams(
            dimension_semantics=("parallel", "arbitrary")),
        name="cs_pairwise",
    )(aug_row, aug_col)

    return jnp.log(jnp.sum(partials) + jnp.sum(bc_parts))


# speedup vs baseline: 2.9597x; 2.4191x over previous
import functools

import numpy as np

import jax
import jax.numpy as jnp
from jax.experimental import pallas as pl
from jax.experimental.pallas import tpu as pltpu

TWO_PI = 2.0 * np.pi


def _aug_kernel(x_ref, means_ref, var_ref, lp_ref, row_ref, col_ref, bc_ref,
                *, c, gamma, n, d, k):
    x = x_ref[...]
    sa = jnp.sum(x * x, axis=1, keepdims=True)
    nrm = c * sa + 0.5
    row_ref[:, 0:64] = x * (-2.0 * c)
    row_ref[:, 64:65] = nrm
    row_ref[:, 65:66] = jnp.ones_like(nrm)
    col_ref[:, 0:64] = x
    col_ref[:, 64:65] = jnp.ones_like(nrm)
    col_ref[:, 65:66] = nrm

    means = means_ref[...]
    var_row = var_ref[...]
    lp_row = lp_ref[...]

    m_gram = jax.lax.dot_general(means, means, (((1,), (1,)), ((), ())),
                                 preferred_element_type=jnp.float32)
    eye = (jax.lax.broadcasted_iota(jnp.int32, (k, k), 0)
           == jax.lax.broadcasted_iota(jnp.int32, (k, k), 1)
           ).astype(jnp.float32)
    sm_col = jnp.sum(m_gram * eye, axis=1, keepdims=True)
    sm_row = jnp.sum(m_gram * eye, axis=0, keepdims=True)

    e = jnp.exp(lp_row - jnp.max(lp_row))
    p_row = e / jnp.sum(e)

    gc = jax.lax.dot_general(x, means, (((1,), (1,)), ((), ())),
                             preferred_element_type=jnp.float32)
    d2c = jnp.maximum(sa + sm_row - 2.0 * gc, 0.0)
    den_c = 2.0 * (var_row + 2.0 * gamma)
    phi_c = jax.lax.rsqrt(1.0 + d2c * ((4.0 / (2 * d - 3)) / den_c))
    coef_c = (2.0 / n) * p_row * jax.lax.rsqrt(
        TWO_PI * (var_row + 2.0 * gamma))
    c_scalar = jnp.sum(coef_c * phi_c)

    var_col = jnp.sum(var_row * eye, axis=1, keepdims=True)
    p_col = jnp.sum(p_row * eye, axis=1, keepdims=True)
    var_mat = var_col + var_row
    b1 = jnp.maximum(sm_col + sm_row - 2.0 * m_gram, 0.0)
    b2 = jax.lax.rsqrt(1.0 + b1 * (4.0 / (2 * d - 3))
                       / (2.0 * var_mat + 4.0 * gamma))
    b3 = p_col * p_row * jax.lax.rsqrt(TWO_PI * (var_mat + 2.0 * gamma))
    b_scalar = jnp.sum(b3 * b2)

    first = jnp.where(pl.program_id(0) == 0, 1.0, 0.0)
    bc_ref[...] = jnp.full((1, 1, 128),
                           (first * b_scalar - c_scalar) / 128.0,
                           dtype=jnp.float32)


def _main_kernel(row_ref, col_ref, out_ref, acc_ref, *, jj_n, scale_a):
    jj = pl.program_id(1)

    @pl.when(jj == 0)
    def _init():
        acc_ref[...] = jnp.zeros_like(acc_ref)

    arg = jax.lax.dot_general(row_ref[...], col_ref[...],
                              (((1,), (1,)), ((), ())),
                              preferred_element_type=jnp.float32)
    phi = jax.lax.rsqrt(jnp.maximum(arg, 1.0))
    red = phi[:, 0:128]
    for g in range(128, phi.shape[1], 128):
        red = red + phi[:, g:g + 128]
    edge = jnp.logical_or(jj == 0, jj == jj_n - 1)
    mult = jnp.where(edge, scale_a, 2.0 * scale_a)
    acc_ref[...] += mult * red

    @pl.when(jj == jj_n - 1)
    def _flush():
        out_ref[...] = jnp.sum(acc_ref[...], axis=0, keepdims=True
                               ).reshape(1, 1, 128)


@jax.jit
def kernel(X, means, variances, logit_probs):
    n, d = X.shape
    k = means.shape[0]
    gamma = float(np.power(4.0 / (3.0 * n / k), 0.4))
    c = 1.0 / ((2 * d - 3) * gamma)
    scale_a = 1.0 / (n * n * np.sqrt(TWO_PI * 2.0 * gamma))

    ti = 1024
    ni = n // ti
    jj_n = ni // 2 + 1

    tb = 2048
    nb = n // tb
    aug_row, aug_col, bc_parts = pl.pallas_call(
        functools.partial(_aug_kernel, c=c, gamma=gamma, n=n, d=d, k=k),
        grid=(nb,),
        in_specs=[pl.BlockSpec((tb, d), lambda b: (b, 0)),
                  pl.BlockSpec((k, d), lambda b: (0, 0)),
                  pl.BlockSpec((1, k), lambda b: (0, 0)),
                  pl.BlockSpec((1, k), lambda b: (0, 0))],
        out_specs=[pl.BlockSpec((tb, 66), lambda b: (b, 0)),
                   pl.BlockSpec((tb, 66), lambda b: (b, 0)),
                   pl.BlockSpec((1, 1, 128), lambda b: (b, 0, 0))],
        out_shape=[jax.ShapeDtypeStruct((n, 66), jnp.float32),
                   jax.ShapeDtypeStruct((n, 66), jnp.float32),
                   jax.ShapeDtypeStruct((nb, 1, 128), jnp.float32)],
        compiler_params=pltpu.CompilerParams(
            dimension_semantics=("arbitrary",)),
        name="cs_aug",
    )(X, means, variances.reshape(1, k), logit_probs.reshape(1, k))

    partials = pl.pallas_call(
        functools.partial(_main_kernel, jj_n=jj_n, scale_a=scale_a),
        grid=(ni, jj_n),
        in_specs=[
            pl.BlockSpec((ti, 66), lambda i, jj: (i, 0)),
            pl.BlockSpec((ti, 66), lambda i, jj: ((i + jj) % ni, 0)),
        ],
        out_specs=pl.BlockSpec((1, 1, 128), lambda i, jj: (i, 0, 0)),
        out_shape=jax.ShapeDtypeStruct((ni, 1, 128), jnp.float32),
        scratch_shapes=[pltpu.VMEM((ti, 128), jnp.float32)],
        compiler_params=pltpu.CompilerParams(
            dimension_semantics=("parallel", "arbitrary")),
        name="cs_pairwise",
    )(aug_row, aug_col)

    return jnp.log(jnp.sum(partials) + jnp.sum(bc_parts))


# speedup vs baseline: 3.0986x; 1.0469x over previous
import functools

import numpy as np

import jax
import jax.numpy as jnp
from jax.experimental import pallas as pl
from jax.experimental.pallas import tpu as pltpu

TWO_PI = 2.0 * np.pi


def _aug_kernel(x_ref, means_ref, var_ref, lp_ref, row_ref, col_ref, bc_ref,
                *, c, gamma, n, d, k):
    x = x_ref[...]
    sa = jnp.sum(x * x, axis=1, keepdims=True)
    nrm = c * sa + 0.5
    row_ref[:, 0:64] = (x * (-2.0 * c)).astype(jnp.bfloat16)
    row_ref[:, 64:65] = nrm.astype(jnp.bfloat16)
    row_ref[:, 65:66] = jnp.ones_like(nrm, dtype=jnp.bfloat16)
    col_ref[:, 0:64] = x.astype(jnp.bfloat16)
    col_ref[:, 64:65] = jnp.ones_like(nrm, dtype=jnp.bfloat16)
    col_ref[:, 65:66] = nrm.astype(jnp.bfloat16)

    means = means_ref[...]
    var_row = var_ref[...]
    lp_row = lp_ref[...]

    m_gram = jax.lax.dot_general(means, means, (((1,), (1,)), ((), ())),
                                 preferred_element_type=jnp.float32)
    eye = (jax.lax.broadcasted_iota(jnp.int32, (k, k), 0)
           == jax.lax.broadcasted_iota(jnp.int32, (k, k), 1)
           ).astype(jnp.float32)
    sm_col = jnp.sum(m_gram * eye, axis=1, keepdims=True)
    sm_row = jnp.sum(m_gram * eye, axis=0, keepdims=True)

    e = jnp.exp(lp_row - jnp.max(lp_row))
    p_row = e / jnp.sum(e)

    gc = jax.lax.dot_general(x, means, (((1,), (1,)), ((), ())),
                             preferred_element_type=jnp.float32)
    d2c = jnp.maximum(sa + sm_row - 2.0 * gc, 0.0)
    den_c = 2.0 * (var_row + 2.0 * gamma)
    phi_c = jax.lax.rsqrt(1.0 + d2c * ((4.0 / (2 * d - 3)) / den_c))
    coef_c = (2.0 / n) * p_row * jax.lax.rsqrt(
        TWO_PI * (var_row + 2.0 * gamma))
    c_scalar = jnp.sum(coef_c * phi_c)

    var_col = jnp.sum(var_row * eye, axis=1, keepdims=True)
    p_col = jnp.sum(p_row * eye, axis=1, keepdims=True)
    var_mat = var_col + var_row
    b1 = jnp.maximum(sm_col + sm_row - 2.0 * m_gram, 0.0)
    b2 = jax.lax.rsqrt(1.0 + b1 * (4.0 / (2 * d - 3))
                       / (2.0 * var_mat + 4.0 * gamma))
    b3 = p_col * p_row * jax.lax.rsqrt(TWO_PI * (var_mat + 2.0 * gamma))
    b_scalar = jnp.sum(b3 * b2)

    first = jnp.where(pl.program_id(0) == 0, 1.0, 0.0)
    bc_ref[...] = jnp.full((1, 1, 128),
                           (first * b_scalar - c_scalar) / 128.0,
                           dtype=jnp.float32)


def _main_kernel(row_ref, col_ref, out_ref, acc_ref, *, jj_n, scale_a):
    jj = pl.program_id(1)

    @pl.when(jj == 0)
    def _init():
        acc_ref[...] = jnp.zeros_like(acc_ref)

    arg = jax.lax.dot_general(row_ref[...], col_ref[...],
                              (((1,), (1,)), ((), ())),
                              preferred_element_type=jnp.float32)
    phi = jax.lax.rsqrt(jnp.maximum(arg, 1.0))
    red = phi[:, 0:128]
    for g in range(128, phi.shape[1], 128):
        red = red + phi[:, g:g + 128]
    edge = jnp.logical_or(jj == 0, jj == jj_n - 1)
    mult = jnp.where(edge, scale_a, 2.0 * scale_a)
    acc_ref[...] += mult * red

    @pl.when(jj == jj_n - 1)
    def _flush():
        out_ref[...] = jnp.sum(acc_ref[...], axis=0, keepdims=True
                               ).reshape(1, 1, 128)


@jax.jit
def kernel(X, means, variances, logit_probs):
    n, d = X.shape
    k = means.shape[0]
    gamma = float(np.power(4.0 / (3.0 * n / k), 0.4))
    c = 1.0 / ((2 * d - 3) * gamma)
    scale_a = 1.0 / (n * n * np.sqrt(TWO_PI * 2.0 * gamma))

    ti = 1024
    ni = n // ti
    jj_n = ni // 2 + 1

    tb = 2048
    nb = n // tb
    aug_row, aug_col, bc_parts = pl.pallas_call(
        functools.partial(_aug_kernel, c=c, gamma=gamma, n=n, d=d, k=k),
        grid=(nb,),
        in_specs=[pl.BlockSpec((tb, d), lambda b: (b, 0)),
                  pl.BlockSpec((k, d), lambda b: (0, 0)),
                  pl.BlockSpec((1, k), lambda b: (0, 0)),
                  pl.BlockSpec((1, k), lambda b: (0, 0))],
        out_specs=[pl.BlockSpec((tb, 66), lambda b: (b, 0)),
                   pl.BlockSpec((tb, 66), lambda b: (b, 0)),
                   pl.BlockSpec((1, 1, 128), lambda b: (b, 0, 0))],
        out_shape=[jax.ShapeDtypeStruct((n, 66), jnp.bfloat16),
                   jax.ShapeDtypeStruct((n, 66), jnp.bfloat16),
                   jax.ShapeDtypeStruct((nb, 1, 128), jnp.float32)],
        compiler_params=pltpu.CompilerParams(
            dimension_semantics=("arbitrary",)),
        name="cs_aug",
    )(X, means, variances.reshape(1, k), logit_probs.reshape(1, k))

    partials = pl.pallas_call(
        functools.partial(_main_kernel, jj_n=jj_n, scale_a=scale_a),
        grid=(ni, jj_n),
        in_specs=[
            pl.BlockSpec((ti, 66), lambda i, jj: (i, 0)),
            pl.BlockSpec((ti, 66), lambda i, jj: ((i + jj) % ni, 0)),
        ],
        out_specs=pl.BlockSpec((1, 1, 128), lambda i, jj: (i, 0, 0)),
        out_shape=jax.ShapeDtypeStruct((ni, 1, 128), jnp.float32),
        scratch_shapes=[pltpu.VMEM((ti, 128), jnp.float32)],
        compiler_params=pltpu.CompilerParams(
            dimension_semantics=("parallel", "arbitrary")),
        name="cs_pairwise",
    )(aug_row, aug_col)

    return jnp.log(jnp.sum(partials) + jnp.sum(bc_parts))


# speedup vs baseline: 3.1287x; 1.0097x over previous
import functools

import numpy as np

import jax
import jax.numpy as jnp
from jax.experimental import pallas as pl
from jax.experimental.pallas import tpu as pltpu

TWO_PI = 2.0 * np.pi


def _aug_kernel(x_ref, means_ref, var_ref, lp_ref, row_ref, col_ref, bc_ref,
                *, c, gamma, n, d, k):
    x = x_ref[...]
    sa = jnp.sum(x * x, axis=1, keepdims=True)
    nrm = c * sa + 0.5
    row_ref[:, 0:64] = (x * (-2.0 * c)).astype(jnp.bfloat16)
    row_ref[:, 64:65] = nrm.astype(jnp.bfloat16)
    row_ref[:, 65:66] = jnp.ones_like(nrm, dtype=jnp.bfloat16)
    col_ref[:, 0:64] = x.astype(jnp.bfloat16)
    col_ref[:, 64:65] = jnp.ones_like(nrm, dtype=jnp.bfloat16)
    col_ref[:, 65:66] = nrm.astype(jnp.bfloat16)

    means = means_ref[...]
    var_row = var_ref[...]
    lp_row = lp_ref[...]

    m_gram = jax.lax.dot_general(means, means, (((1,), (1,)), ((), ())),
                                 preferred_element_type=jnp.float32)
    eye = (jax.lax.broadcasted_iota(jnp.int32, (k, k), 0)
           == jax.lax.broadcasted_iota(jnp.int32, (k, k), 1)
           ).astype(jnp.float32)
    sm_col = jnp.sum(m_gram * eye, axis=1, keepdims=True)
    sm_row = jnp.sum(m_gram * eye, axis=0, keepdims=True)

    e = jnp.exp(lp_row - jnp.max(lp_row))
    p_row = e / jnp.sum(e)

    gc = jax.lax.dot_general(x, means, (((1,), (1,)), ((), ())),
                             preferred_element_type=jnp.float32)
    d2c = jnp.maximum(sa + sm_row - 2.0 * gc, 0.0)
    den_c = 2.0 * (var_row + 2.0 * gamma)
    phi_c = jax.lax.rsqrt(1.0 + d2c * ((4.0 / (2 * d - 3)) / den_c))
    coef_c = (2.0 / n) * p_row * jax.lax.rsqrt(
        TWO_PI * (var_row + 2.0 * gamma))
    c_scalar = jnp.sum(coef_c * phi_c)

    var_col = jnp.sum(var_row * eye, axis=1, keepdims=True)
    p_col = jnp.sum(p_row * eye, axis=1, keepdims=True)
    var_mat = var_col + var_row
    b1 = jnp.maximum(sm_col + sm_row - 2.0 * m_gram, 0.0)
    b2 = jax.lax.rsqrt(1.0 + b1 * (4.0 / (2 * d - 3))
                       / (2.0 * var_mat + 4.0 * gamma))
    b3 = p_col * p_row * jax.lax.rsqrt(TWO_PI * (var_mat + 2.0 * gamma))
    b_scalar = jnp.sum(b3 * b2)

    first = jnp.where(pl.program_id(0) == 0, 1.0, 0.0)
    bc_ref[...] = jnp.full((1, 1, 128),
                           (first * b_scalar - c_scalar) / 128.0,
                           dtype=jnp.float32)


def _main_kernel(row_ref, col_ref, out_ref, acc_ref, *, jj_n, scale_a):
    jj = pl.program_id(1)

    @pl.when(jj == 0)
    def _init():
        acc_ref[...] = jnp.zeros_like(acc_ref)

    arg = jax.lax.dot_general(row_ref[...], col_ref[...],
                              (((1,), (1,)), ((), ())),
                              preferred_element_type=jnp.float32)
    phi = jax.lax.rsqrt(jnp.maximum(arg, 1.0).astype(jnp.bfloat16))
    red = phi[:, 0:128]
    for g in range(128, phi.shape[1], 128):
        red = red + phi[:, g:g + 128]
    edge = jnp.logical_or(jj == 0, jj == jj_n - 1)
    mult = jnp.where(edge, scale_a, 2.0 * scale_a)
    acc_ref[...] += mult * red.astype(jnp.float32)

    @pl.when(jj == jj_n - 1)
    def _flush():
        out_ref[...] = jnp.sum(acc_ref[...], axis=0, keepdims=True
                               ).reshape(1, 1, 128)


@jax.jit
def kernel(X, means, variances, logit_probs):
    n, d = X.shape
    k = means.shape[0]
    gamma = float(np.power(4.0 / (3.0 * n / k), 0.4))
    c = 1.0 / ((2 * d - 3) * gamma)
    scale_a = 1.0 / (n * n * np.sqrt(TWO_PI * 2.0 * gamma))

    ti = 1024
    ni = n // ti
    jj_n = ni // 2 + 1

    tb = 2048
    nb = n // tb
    aug_row, aug_col, bc_parts = pl.pallas_call(
        functools.partial(_aug_kernel, c=c, gamma=gamma, n=n, d=d, k=k),
        grid=(nb,),
        in_specs=[pl.BlockSpec((tb, d), lambda b: (b, 0)),
                  pl.BlockSpec((k, d), lambda b: (0, 0)),
                  pl.BlockSpec((1, k), lambda b: (0, 0)),
                  pl.BlockSpec((1, k), lambda b: (0, 0))],
        out_specs=[pl.BlockSpec((tb, 66), lambda b: (b, 0)),
                   pl.BlockSpec((tb, 66), lambda b: (b, 0)),
                   pl.BlockSpec((1, 1, 128), lambda b: (b, 0, 0))],
        out_shape=[jax.ShapeDtypeStruct((n, 66), jnp.bfloat16),
                   jax.ShapeDtypeStruct((n, 66), jnp.bfloat16),
                   jax.ShapeDtypeStruct((nb, 1, 128), jnp.float32)],
        compiler_params=pltpu.CompilerParams(
            dimension_semantics=("arbitrary",)),
        name="cs_aug",
    )(X, means, variances.reshape(1, k), logit_probs.reshape(1, k))

    partials = pl.pallas_call(
        functools.partial(_main_kernel, jj_n=jj_n, scale_a=scale_a),
        grid=(ni, jj_n),
        in_specs=[
            pl.BlockSpec((ti, 66), lambda i, jj: (i, 0)),
            pl.BlockSpec((ti, 66), lambda i, jj: ((i + jj) % ni, 0)),
        ],
        out_specs=pl.BlockSpec((1, 1, 128), lambda i, jj: (i, 0, 0)),
        out_shape=jax.ShapeDtypeStruct((ni, 1, 128), jnp.float32),
        scratch_shapes=[pltpu.VMEM((ti, 128), jnp.float32)],
        compiler_params=pltpu.CompilerParams(
            dimension_semantics=("parallel", "arbitrary")),
        name="cs_pairwise",
    )(aug_row, aug_col)

    return jnp.log(jnp.sum(partials) + jnp.sum(bc_parts))


# speedup vs baseline: 3.8617x; 1.2343x over previous
import functools

import numpy as np

import jax
import jax.numpy as jnp
from jax.experimental import pallas as pl
from jax.experimental.pallas import tpu as pltpu

TWO_PI = 2.0 * np.pi


def _aug_kernel(x_ref, means_ref, var_ref, lp_ref, row_ref, col_ref, bc_ref,
                *, c, gamma, n, d, k):
    x = x_ref[...]
    sa = jnp.sum(x * x, axis=1, keepdims=True)
    nrm = c * sa + 0.5
    row_ref[:, 0:64] = (x * (-2.0 * c)).astype(jnp.bfloat16)
    row_ref[:, 64:65] = nrm.astype(jnp.bfloat16)
    row_ref[:, 65:66] = jnp.ones_like(nrm, dtype=jnp.bfloat16)
    col_ref[:, 0:64] = x.astype(jnp.bfloat16)
    col_ref[:, 64:65] = jnp.ones_like(nrm, dtype=jnp.bfloat16)
    col_ref[:, 65:66] = nrm.astype(jnp.bfloat16)

    means = means_ref[...]
    var_row = var_ref[...]
    lp_row = lp_ref[...]

    m_gram = jax.lax.dot_general(means, means, (((1,), (1,)), ((), ())),
                                 preferred_element_type=jnp.float32)
    eye = (jax.lax.broadcasted_iota(jnp.int32, (k, k), 0)
           == jax.lax.broadcasted_iota(jnp.int32, (k, k), 1)
           ).astype(jnp.float32)
    sm_col = jnp.sum(m_gram * eye, axis=1, keepdims=True)
    sm_row = jnp.sum(m_gram * eye, axis=0, keepdims=True)

    e = jnp.exp(lp_row - jnp.max(lp_row))
    p_row = e / jnp.sum(e)

    gc = jax.lax.dot_general(x, means, (((1,), (1,)), ((), ())),
                             preferred_element_type=jnp.float32)
    d2c = jnp.maximum(sa + sm_row - 2.0 * gc, 0.0)
    den_c = 2.0 * (var_row + 2.0 * gamma)
    phi_c = jax.lax.rsqrt(1.0 + d2c * ((4.0 / (2 * d - 3)) / den_c))
    coef_c = (2.0 / n) * p_row * jax.lax.rsqrt(
        TWO_PI * (var_row + 2.0 * gamma))
    c_scalar = jnp.sum(coef_c * phi_c)

    var_col = jnp.sum(var_row * eye, axis=1, keepdims=True)
    p_col = jnp.sum(p_row * eye, axis=1, keepdims=True)
    var_mat = var_col + var_row
    b1 = jnp.maximum(sm_col + sm_row - 2.0 * m_gram, 0.0)
    b2 = jax.lax.rsqrt(1.0 + b1 * (4.0 / (2 * d - 3))
                       / (2.0 * var_mat + 4.0 * gamma))
    b3 = p_col * p_row * jax.lax.rsqrt(TWO_PI * (var_mat + 2.0 * gamma))
    b_scalar = jnp.sum(b3 * b2)

    first = jnp.where(pl.program_id(0) == 0, 1.0, 0.0)
    bc_ref[...] = jnp.full((1, 1, 128),
                           (first * b_scalar - c_scalar) / 128.0,
                           dtype=jnp.float32)


def _main_kernel(row_ref, col_ref, out_ref, acc_ref, *, jj_n, scale_a):
    jj = pl.program_id(1)

    @pl.when(jj == 0)
    def _init():
        acc_ref[...] = jnp.zeros_like(acc_ref)

    arg = jax.lax.dot_general(row_ref[...], col_ref[...],
                              (((1,), (1,)), ((), ())),
                              preferred_element_type=jnp.float32)
    phi = jax.lax.rsqrt(jnp.maximum(arg, 1.0).astype(jnp.bfloat16))
    red = phi[:, 0:128]
    for g in range(128, phi.shape[1], 128):
        red = red + phi[:, g:g + 128]
    edge = jnp.logical_or(jj == 0, jj == jj_n - 1)
    mult = jnp.where(edge, scale_a, 2.0 * scale_a)
    acc_ref[...] += mult * red.astype(jnp.float32)

    @pl.when(jj == jj_n - 1)
    def _flush():
        out_ref[...] = jnp.sum(acc_ref[...], axis=0, keepdims=True
                               ).reshape(1, 1, 128)


@jax.jit
def kernel(X, means, variances, logit_probs):
    n, d = X.shape
    k = means.shape[0]
    gamma = float(np.power(4.0 / (3.0 * n / k), 0.4))
    c = 1.0 / ((2 * d - 3) * gamma)
    scale_a = 1.0 / (n * n * np.sqrt(TWO_PI * 2.0 * gamma))

    ti = 2048
    ni = n // ti
    jj_n = ni // 2 + 1

    tb = 2048
    nb = n // tb
    aug_row, aug_col, bc_parts = pl.pallas_call(
        functools.partial(_aug_kernel, c=c, gamma=gamma, n=n, d=d, k=k),
        grid=(nb,),
        in_specs=[pl.BlockSpec((tb, d), lambda b: (b, 0)),
                  pl.BlockSpec((k, d), lambda b: (0, 0)),
                  pl.BlockSpec((1, k), lambda b: (0, 0)),
                  pl.BlockSpec((1, k), lambda b: (0, 0))],
        out_specs=[pl.BlockSpec((tb, 66), lambda b: (b, 0)),
                   pl.BlockSpec((tb, 66), lambda b: (b, 0)),
                   pl.BlockSpec((1, 1, 128), lambda b: (b, 0, 0))],
        out_shape=[jax.ShapeDtypeStruct((n, 66), jnp.bfloat16),
                   jax.ShapeDtypeStruct((n, 66), jnp.bfloat16),
                   jax.ShapeDtypeStruct((nb, 1, 128), jnp.float32)],
        compiler_params=pltpu.CompilerParams(
            dimension_semantics=("arbitrary",)),
        name="cs_aug",
    )(X, means, variances.reshape(1, k), logit_probs.reshape(1, k))

    partials = pl.pallas_call(
        functools.partial(_main_kernel, jj_n=jj_n, scale_a=scale_a),
        grid=(ni, jj_n),
        in_specs=[
            pl.BlockSpec((ti, 66), lambda i, jj: (i, 0)),
            pl.BlockSpec((ti, 66), lambda i, jj: ((i + jj) % ni, 0)),
        ],
        out_specs=pl.BlockSpec((1, 1, 128), lambda i, jj: (i, 0, 0)),
        out_shape=jax.ShapeDtypeStruct((ni, 1, 128), jnp.float32),
        scratch_shapes=[pltpu.VMEM((ti, 128), jnp.float32)],
        compiler_params=pltpu.CompilerParams(
            dimension_semantics=("parallel", "arbitrary")),
        name="cs_pairwise",
    )(aug_row, aug_col)

    return jnp.log(jnp.sum(partials) + jnp.sum(bc_parts))


# speedup vs baseline: 4.0080x; 1.0379x over previous
import functools

import numpy as np

import jax
import jax.numpy as jnp
from jax.experimental import pallas as pl
from jax.experimental.pallas import tpu as pltpu

TWO_PI = 2.0 * np.pi


def _aug_kernel(x_ref, means_ref, var_ref, lp_ref, row_ref, col_ref, bc_ref,
                *, c, gamma, n, d, k):
    f8 = jnp.float8_e4m3fn
    x = x_ref[...]
    sa = jnp.sum(x * x, axis=1, keepdims=True)
    nrm = c * sa + 0.5
    n1 = nrm.astype(f8)
    n2 = (nrm - n1.astype(jnp.float32)).astype(f8)
    one = jnp.ones_like(nrm, dtype=f8)
    row_ref[:, 0:64] = (x * (-2.0 * c)).astype(f8)
    row_ref[:, 64:65] = n1
    row_ref[:, 65:66] = n2
    row_ref[:, 66:67] = one
    row_ref[:, 67:68] = one
    col_ref[:, 0:64] = x.astype(f8)
    col_ref[:, 64:65] = one
    col_ref[:, 65:66] = one
    col_ref[:, 66:67] = n1
    col_ref[:, 67:68] = n2

    means = means_ref[...]
    var_row = var_ref[...]
    lp_row = lp_ref[...]

    m_gram = jax.lax.dot_general(means, means, (((1,), (1,)), ((), ())),
                                 preferred_element_type=jnp.float32)
    eye = (jax.lax.broadcasted_iota(jnp.int32, (k, k), 0)
           == jax.lax.broadcasted_iota(jnp.int32, (k, k), 1)
           ).astype(jnp.float32)
    sm_col = jnp.sum(m_gram * eye, axis=1, keepdims=True)
    sm_row = jnp.sum(m_gram * eye, axis=0, keepdims=True)

    e = jnp.exp(lp_row - jnp.max(lp_row))
    p_row = e / jnp.sum(e)

    gc = jax.lax.dot_general(x, means, (((1,), (1,)), ((), ())),
                             preferred_element_type=jnp.float32)
    d2c = jnp.maximum(sa + sm_row - 2.0 * gc, 0.0)
    den_c = 2.0 * (var_row + 2.0 * gamma)
    phi_c = jax.lax.rsqrt(1.0 + d2c * ((4.0 / (2 * d - 3)) / den_c))
    coef_c = (2.0 / n) * p_row * jax.lax.rsqrt(
        TWO_PI * (var_row + 2.0 * gamma))
    c_scalar = jnp.sum(coef_c * phi_c)

    var_col = jnp.sum(var_row * eye, axis=1, keepdims=True)
    p_col = jnp.sum(p_row * eye, axis=1, keepdims=True)
    var_mat = var_col + var_row
    b1 = jnp.maximum(sm_col + sm_row - 2.0 * m_gram, 0.0)
    b2 = jax.lax.rsqrt(1.0 + b1 * (4.0 / (2 * d - 3))
                       / (2.0 * var_mat + 4.0 * gamma))
    b3 = p_col * p_row * jax.lax.rsqrt(TWO_PI * (var_mat + 2.0 * gamma))
    b_scalar = jnp.sum(b3 * b2)

    first = jnp.where(pl.program_id(0) == 0, 1.0, 0.0)
    bc_ref[...] = jnp.full((1, 1, 128),
                           (first * b_scalar - c_scalar) / 128.0,
                           dtype=jnp.float32)


def _main_kernel(row_ref, col_ref, out_ref, acc_ref, *, jj_n, scale_a):
    jj = pl.program_id(1)

    @pl.when(jj == 0)
    def _init():
        acc_ref[...] = jnp.zeros_like(acc_ref)

    rows = row_ref[...]
    red = None
    for c0 in range(0, col_ref.shape[0], 256):
        arg = jax.lax.dot_general(rows, col_ref[c0:c0 + 256, :],
                                  (((1,), (1,)), ((), ())),
                                  preferred_element_type=jnp.float32)
        phi = jax.lax.rsqrt(jnp.maximum(arg.astype(jnp.bfloat16), 1.0))
        h = phi[:, 0:128] + phi[:, 128:256]
        red = h if red is None else red + h
    edge = jnp.logical_or(jj == 0, jj == jj_n - 1)
    mult = jnp.where(edge, scale_a, 2.0 * scale_a)
    acc_ref[...] += mult * red.astype(jnp.float32)

    @pl.when(jj == jj_n - 1)
    def _flush():
        out_ref[...] = jnp.sum(acc_ref[...], axis=0, keepdims=True
                               ).reshape(1, 1, 128)


@jax.jit
def kernel(X, means, variances, logit_probs):
    n, d = X.shape
    k = means.shape[0]
    gamma = float(np.power(4.0 / (3.0 * n / k), 0.4))
    c = 1.0 / ((2 * d - 3) * gamma)
    scale_a = 1.0 / (n * n * np.sqrt(TWO_PI * 2.0 * gamma))

    ti = 2048
    ni = n // ti
    jj_n = ni // 2 + 1

    tb = 2048
    nb = n // tb
    aug_row, aug_col, bc_parts = pl.pallas_call(
        functools.partial(_aug_kernel, c=c, gamma=gamma, n=n, d=d, k=k),
        grid=(nb,),
        in_specs=[pl.BlockSpec((tb, d), lambda b: (b, 0)),
                  pl.BlockSpec((k, d), lambda b: (0, 0)),
                  pl.BlockSpec((1, k), lambda b: (0, 0)),
                  pl.BlockSpec((1, k), lambda b: (0, 0))],
        out_specs=[pl.BlockSpec((tb, 68), lambda b: (b, 0)),
                   pl.BlockSpec((tb, 68), lambda b: (b, 0)),
                   pl.BlockSpec((1, 1, 128), lambda b: (b, 0, 0))],
        out_shape=[jax.ShapeDtypeStruct((n, 68), jnp.float8_e4m3fn),
                   jax.ShapeDtypeStruct((n, 68), jnp.float8_e4m3fn),
                   jax.ShapeDtypeStruct((nb, 1, 128), jnp.float32)],
        compiler_params=pltpu.CompilerParams(
            dimension_semantics=("arbitrary",)),
        name="cs_aug",
    )(X, means, variances.reshape(1, k), logit_probs.reshape(1, k))

    partials = pl.pallas_call(
        functools.partial(_main_kernel, jj_n=jj_n, scale_a=scale_a),
        grid=(ni, jj_n),
        in_specs=[
            pl.BlockSpec((ti, 68), lambda i, jj: (i, 0)),
            pl.BlockSpec((ti, 68), lambda i, jj: ((i + jj) % ni, 0)),
        ],
        out_specs=pl.BlockSpec((1, 1, 128), lambda i, jj: (i, 0, 0)),
        out_shape=jax.ShapeDtypeStruct((ni, 1, 128), jnp.float32),
        scratch_shapes=[pltpu.VMEM((ti, 128), jnp.float32)],
        compiler_params=pltpu.CompilerParams(
            dimension_semantics=("parallel", "arbitrary")),
        name="cs_pairwise",
    )(aug_row, aug_col)

    return jnp.log(jnp.sum(partials) + jnp.sum(bc_parts))


# speedup vs baseline: 4.0084x; 1.0001x over previous
import functools

import numpy as np

import jax
import jax.numpy as jnp
from jax.experimental import pallas as pl
from jax.experimental.pallas import tpu as pltpu

TWO_PI = 2.0 * np.pi


def _aug_kernel(x_ref, means_ref, var_ref, lp_ref, row_ref, col_ref, bc_ref,
                *, c, gamma, n, d, k):
    f8 = jnp.float8_e4m3fn
    x = x_ref[...]
    sa = jnp.sum(x * x, axis=1, keepdims=True)
    nrm = c * sa + 0.5
    n1 = nrm.astype(f8)
    n2 = (nrm - n1.astype(jnp.float32)).astype(f8)
    one = jnp.ones_like(nrm, dtype=f8)
    row_ref[:, 0:64] = (x * (-2.0 * c)).astype(f8)
    row_ref[:, 64:65] = n1
    row_ref[:, 65:66] = n2
    row_ref[:, 66:67] = one
    row_ref[:, 67:68] = one
    col_ref[:, 0:64] = x.astype(f8)
    col_ref[:, 64:65] = one
    col_ref[:, 65:66] = one
    col_ref[:, 66:67] = n1
    col_ref[:, 67:68] = n2

    means = means_ref[...]
    var_row = var_ref[...]
    lp_row = lp_ref[...]

    m_gram = jax.lax.dot_general(means, means, (((1,), (1,)), ((), ())),
                                 preferred_element_type=jnp.float32)
    eye = (jax.lax.broadcasted_iota(jnp.int32, (k, k), 0)
           == jax.lax.broadcasted_iota(jnp.int32, (k, k), 1)
           ).astype(jnp.float32)
    sm_col = jnp.sum(m_gram * eye, axis=1, keepdims=True)
    sm_row = jnp.sum(m_gram * eye, axis=0, keepdims=True)

    e = jnp.exp(lp_row - jnp.max(lp_row))
    p_row = e / jnp.sum(e)

    gc = jax.lax.dot_general(x, means, (((1,), (1,)), ((), ())),
                             preferred_element_type=jnp.float32)
    d2c = jnp.maximum(sa + sm_row - 2.0 * gc, 0.0)
    den_c = 2.0 * (var_row + 2.0 * gamma)
    phi_c = jax.lax.rsqrt(1.0 + d2c * ((4.0 / (2 * d - 3)) / den_c))
    coef_c = (2.0 / n) * p_row * jax.lax.rsqrt(
        TWO_PI * (var_row + 2.0 * gamma))
    c_scalar = jnp.sum(coef_c * phi_c)

    var_col = jnp.sum(var_row * eye, axis=1, keepdims=True)
    p_col = jnp.sum(p_row * eye, axis=1, keepdims=True)
    var_mat = var_col + var_row
    b1 = jnp.maximum(sm_col + sm_row - 2.0 * m_gram, 0.0)
    b2 = jax.lax.rsqrt(1.0 + b1 * (4.0 / (2 * d - 3))
                       / (2.0 * var_mat + 4.0 * gamma))
    b3 = p_col * p_row * jax.lax.rsqrt(TWO_PI * (var_mat + 2.0 * gamma))
    b_scalar = jnp.sum(b3 * b2)

    first = jnp.where(pl.program_id(0) == 0, 1.0, 0.0)
    bc_ref[...] = jnp.full((1, 1, 128),
                           (first * b_scalar - c_scalar) / 128.0,
                           dtype=jnp.float32)


def _main_kernel(row_ref, col_ref, out_ref, acc_ref, *, jj_n, scale_a):
    jj = pl.program_id(1)

    @pl.when(jj == 0)
    def _init():
        acc_ref[...] = jnp.zeros_like(acc_ref)

    rows = row_ref[...]
    red = None
    for c0 in range(0, col_ref.shape[0], 512):
        arg = jax.lax.dot_general(rows, col_ref[c0:c0 + 512, :],
                                  (((1,), (1,)), ((), ())),
                                  preferred_element_type=jnp.float32)
        phi = jax.lax.rsqrt(jnp.maximum(arg.astype(jnp.bfloat16), 1.0))
        h = (phi[:, 0:128] + phi[:, 128:256]) + (phi[:, 256:384]
                                                 + phi[:, 384:512])
        red = h if red is None else red + h
    edge = jnp.logical_or(jj == 0, jj == jj_n - 1)
    mult = jnp.where(edge, scale_a, 2.0 * scale_a)
    acc_ref[...] += mult * red.astype(jnp.float32)

    @pl.when(jj == jj_n - 1)
    def _flush():
        out_ref[...] = jnp.sum(acc_ref[...], axis=0, keepdims=True
                               ).reshape(1, 1, 128)


@jax.jit
def kernel(X, means, variances, logit_probs):
    n, d = X.shape
    k = means.shape[0]
    gamma = float(np.power(4.0 / (3.0 * n / k), 0.4))
    c = 1.0 / ((2 * d - 3) * gamma)
    scale_a = 1.0 / (n * n * np.sqrt(TWO_PI * 2.0 * gamma))

    ti = 2048
    ni = n // ti
    jj_n = ni // 2 + 1

    tb = 2048
    nb = n // tb
    aug_row, aug_col, bc_parts = pl.pallas_call(
        functools.partial(_aug_kernel, c=c, gamma=gamma, n=n, d=d, k=k),
        grid=(nb,),
        in_specs=[pl.BlockSpec((tb, d), lambda b: (b, 0)),
                  pl.BlockSpec((k, d), lambda b: (0, 0)),
                  pl.BlockSpec((1, k), lambda b: (0, 0)),
                  pl.BlockSpec((1, k), lambda b: (0, 0))],
        out_specs=[pl.BlockSpec((tb, 68), lambda b: (b, 0)),
                   pl.BlockSpec((tb, 68), lambda b: (b, 0)),
                   pl.BlockSpec((1, 1, 128), lambda b: (b, 0, 0))],
        out_shape=[jax.ShapeDtypeStruct((n, 68), jnp.float8_e4m3fn),
                   jax.ShapeDtypeStruct((n, 68), jnp.float8_e4m3fn),
                   jax.ShapeDtypeStruct((nb, 1, 128), jnp.float32)],
        compiler_params=pltpu.CompilerParams(
            dimension_semantics=("arbitrary",)),
        name="cs_aug",
    )(X, means, variances.reshape(1, k), logit_probs.reshape(1, k))

    partials = pl.pallas_call(
        functools.partial(_main_kernel, jj_n=jj_n, scale_a=scale_a),
        grid=(ni, jj_n),
        in_specs=[
            pl.BlockSpec((ti, 68), lambda i, jj: (i, 0)),
            pl.BlockSpec((ti, 68), lambda i, jj: ((i + jj) % ni, 0)),
        ],
        out_specs=pl.BlockSpec((1, 1, 128), lambda i, jj: (i, 0, 0)),
        out_shape=jax.ShapeDtypeStruct((ni, 1, 128), jnp.float32),
        scratch_shapes=[pltpu.VMEM((ti, 128), jnp.float32)],
        compiler_params=pltpu.CompilerParams(
            dimension_semantics=("parallel", "arbitrary")),
        name="cs_pairwise",
    )(aug_row, aug_col)

    return jnp.log(jnp.sum(partials) + jnp.sum(bc_parts))


# speedup vs baseline: 4.2302x; 1.0553x over previous
import functools

import numpy as np

import jax
import jax.numpy as jnp
from jax.experimental import pallas as pl
from jax.experimental.pallas import tpu as pltpu

TWO_PI = 2.0 * np.pi


def _aug_kernel(x_ref, means_ref, var_ref, lp_ref, row_ref, col_ref, bc_ref,
                *, c, gamma, n, d, k):
    f8 = jnp.float8_e4m3fn
    x = x_ref[...]
    sa = jnp.sum(x * x, axis=1, keepdims=True)
    nrm = c * sa + 0.5
    n1 = nrm.astype(f8)
    n2 = (nrm - n1.astype(jnp.float32)).astype(f8)
    one = jnp.ones_like(nrm, dtype=f8)
    row_ref[:, 0:64] = (x * (-2.0 * c)).astype(f8)
    row_ref[:, 64:65] = n1
    row_ref[:, 65:66] = n2
    row_ref[:, 66:67] = one
    row_ref[:, 67:68] = one
    col_ref[:, 0:64] = x.astype(f8)
    col_ref[:, 64:65] = one
    col_ref[:, 65:66] = one
    col_ref[:, 66:67] = n1
    col_ref[:, 67:68] = n2

    means = means_ref[...]
    var_row = var_ref[...]
    lp_row = lp_ref[...]

    m_gram = jax.lax.dot_general(means, means, (((1,), (1,)), ((), ())),
                                 preferred_element_type=jnp.float32)
    eye = (jax.lax.broadcasted_iota(jnp.int32, (k, k), 0)
           == jax.lax.broadcasted_iota(jnp.int32, (k, k), 1)
           ).astype(jnp.float32)
    sm_col = jnp.sum(m_gram * eye, axis=1, keepdims=True)
    sm_row = jnp.sum(m_gram * eye, axis=0, keepdims=True)

    e = jnp.exp(lp_row - jnp.max(lp_row))
    p_row = e / jnp.sum(e)

    gc = jax.lax.dot_general(x, means, (((1,), (1,)), ((), ())),
                             preferred_element_type=jnp.float32)
    d2c = jnp.maximum(sa + sm_row - 2.0 * gc, 0.0)
    den_c = 2.0 * (var_row + 2.0 * gamma)
    phi_c = jax.lax.rsqrt(1.0 + d2c * ((4.0 / (2 * d - 3)) / den_c))
    coef_c = (2.0 / n) * p_row * jax.lax.rsqrt(
        TWO_PI * (var_row + 2.0 * gamma))
    c_scalar = jnp.sum(coef_c * phi_c)

    var_col = jnp.sum(var_row * eye, axis=1, keepdims=True)
    p_col = jnp.sum(p_row * eye, axis=1, keepdims=True)
    var_mat = var_col + var_row
    b1 = jnp.maximum(sm_col + sm_row - 2.0 * m_gram, 0.0)
    b2 = jax.lax.rsqrt(1.0 + b1 * (4.0 / (2 * d - 3))
                       / (2.0 * var_mat + 4.0 * gamma))
    b3 = p_col * p_row * jax.lax.rsqrt(TWO_PI * (var_mat + 2.0 * gamma))
    b_scalar = jnp.sum(b3 * b2)

    first = jnp.where(pl.program_id(0) == 0, 1.0, 0.0)
    bc_ref[...] = jnp.full((1, 1, 128),
                           (first * b_scalar - c_scalar) / 128.0,
                           dtype=jnp.float32)


def _main_kernel(row_ref, col_ref, out_ref, acc_ref, *, jj_n, scale_a):
    i = pl.program_id(0)
    jj = pl.program_id(1)
    ni = pl.num_programs(0)

    @pl.when(jj == 0)
    def _init():
        acc_ref[...] = jnp.zeros_like(acc_ref)

    active = jnp.logical_or(jj < jj_n - 1, i < ni // 2)

    @pl.when(active)
    def _compute():
        rows = row_ref[...]
        red = None
        for c0 in range(0, col_ref.shape[0], 512):
            arg = jax.lax.dot_general(rows, col_ref[c0:c0 + 512, :],
                                      (((1,), (1,)), ((), ())),
                                      preferred_element_type=jnp.float32)
            phi = jax.lax.rsqrt(jnp.maximum(arg.astype(jnp.bfloat16), 1.0))
            h = (phi[:, 0:128] + phi[:, 128:256]) + (phi[:, 256:384]
                                                     + phi[:, 384:512])
            red = h if red is None else red + h
        mult = jnp.where(jj == 0, scale_a, 2.0 * scale_a)
        acc_ref[...] += mult * red.astype(jnp.float32)

    @pl.when(jj == jj_n - 1)
    def _flush():
        out_ref[...] = jnp.sum(acc_ref[...], axis=0, keepdims=True
                               ).reshape(1, 1, 128)


@jax.jit
def kernel(X, means, variances, logit_probs):
    n, d = X.shape
    k = means.shape[0]
    gamma = float(np.power(4.0 / (3.0 * n / k), 0.4))
    c = 1.0 / ((2 * d - 3) * gamma)
    scale_a = 1.0 / (n * n * np.sqrt(TWO_PI * 2.0 * gamma))

    ti = 2048
    ni = n // ti
    jj_n = ni // 2 + 1

    tb = 2048
    nb = n // tb
    aug_row, aug_col, bc_parts = pl.pallas_call(
        functools.partial(_aug_kernel, c=c, gamma=gamma, n=n, d=d, k=k),
        grid=(nb,),
        in_specs=[pl.BlockSpec((tb, d), lambda b: (b, 0)),
                  pl.BlockSpec((k, d), lambda b: (0, 0)),
                  pl.BlockSpec((1, k), lambda b: (0, 0)),
                  pl.BlockSpec((1, k), lambda b: (0, 0))],
        out_specs=[pl.BlockSpec((tb, 68), lambda b: (b, 0)),
                   pl.BlockSpec((tb, 68), lambda b: (b, 0)),
                   pl.BlockSpec((1, 1, 128), lambda b: (b, 0, 0))],
        out_shape=[jax.ShapeDtypeStruct((n, 68), jnp.float8_e4m3fn),
                   jax.ShapeDtypeStruct((n, 68), jnp.float8_e4m3fn),
                   jax.ShapeDtypeStruct((nb, 1, 128), jnp.float32)],
        compiler_params=pltpu.CompilerParams(
            dimension_semantics=("arbitrary",)),
        name="cs_aug",
    )(X, means, variances.reshape(1, k), logit_probs.reshape(1, k))

    partials = pl.pallas_call(
        functools.partial(_main_kernel, jj_n=jj_n, scale_a=scale_a),
        grid=(ni, jj_n),
        in_specs=[
            pl.BlockSpec((ti, 68), lambda i, jj: (i, 0)),
            pl.BlockSpec((ti, 68), lambda i, jj: ((i + jj) % ni, 0)),
        ],
        out_specs=pl.BlockSpec((1, 1, 128), lambda i, jj: (i, 0, 0)),
        out_shape=jax.ShapeDtypeStruct((ni, 1, 128), jnp.float32),
        scratch_shapes=[pltpu.VMEM((ti, 128), jnp.float32)],
        compiler_params=pltpu.CompilerParams(
            dimension_semantics=("parallel", "arbitrary")),
        name="cs_pairwise",
    )(aug_row, aug_col)

    return jnp.log(jnp.sum(partials) + jnp.sum(bc_parts))


# speedup vs baseline: 4.5041x; 1.0647x over previous
import functools

import numpy as np

import jax
import jax.numpy as jnp
from jax.experimental import pallas as pl
from jax.experimental.pallas import tpu as pltpu

TWO_PI = 2.0 * np.pi


def _aug_kernel(x_ref, means_ref, var_ref, lp_ref, row_ref, col_ref, bc_ref,
                *, c, gamma, n, d, k):
    f8 = jnp.float8_e4m3fn
    x = x_ref[...]
    sa = jnp.sum(x * x, axis=1, keepdims=True)
    nrm = c * sa + 0.5
    n1 = nrm.astype(f8)
    n2 = (nrm - n1.astype(jnp.float32)).astype(f8)
    one = jnp.ones_like(nrm, dtype=f8)
    row_ref[:, 0:64] = (x * (-2.0 * c)).astype(f8)
    row_ref[:, 64:65] = n1
    row_ref[:, 65:66] = n2
    row_ref[:, 66:67] = one
    row_ref[:, 67:68] = one
    col_ref[:, 0:64] = x.astype(f8)
    col_ref[:, 64:65] = one
    col_ref[:, 65:66] = one
    col_ref[:, 66:67] = n1
    col_ref[:, 67:68] = n2

    means = means_ref[...]
    var_row = var_ref[...]
    lp_row = lp_ref[...]

    m_gram = jax.lax.dot_general(means, means, (((1,), (1,)), ((), ())),
                                 preferred_element_type=jnp.float32)
    eye = (jax.lax.broadcasted_iota(jnp.int32, (k, k), 0)
           == jax.lax.broadcasted_iota(jnp.int32, (k, k), 1)
           ).astype(jnp.float32)
    sm_col = jnp.sum(m_gram * eye, axis=1, keepdims=True)
    sm_row = jnp.sum(m_gram * eye, axis=0, keepdims=True)

    e = jnp.exp(lp_row - jnp.max(lp_row))
    p_row = e / jnp.sum(e)

    gc = jax.lax.dot_general(x, means, (((1,), (1,)), ((), ())),
                             preferred_element_type=jnp.float32)
    d2c = jnp.maximum(sa + sm_row - 2.0 * gc, 0.0)
    den_c = 2.0 * (var_row + 2.0 * gamma)
    phi_c = jax.lax.rsqrt(1.0 + d2c * ((4.0 / (2 * d - 3)) / den_c))
    coef_c = (2.0 / n) * p_row * jax.lax.rsqrt(
        TWO_PI * (var_row + 2.0 * gamma))
    c_scalar = jnp.sum(coef_c * phi_c)

    var_col = jnp.sum(var_row * eye, axis=1, keepdims=True)
    p_col = jnp.sum(p_row * eye, axis=1, keepdims=True)
    var_mat = var_col + var_row
    b1 = jnp.maximum(sm_col + sm_row - 2.0 * m_gram, 0.0)
    b2 = jax.lax.rsqrt(1.0 + b1 * (4.0 / (2 * d - 3))
                       / (2.0 * var_mat + 4.0 * gamma))
    b3 = p_col * p_row * jax.lax.rsqrt(TWO_PI * (var_mat + 2.0 * gamma))
    b_scalar = jnp.sum(b3 * b2)

    first = jnp.where(pl.program_id(0) == 0, 1.0, 0.0)
    bc_ref[...] = jnp.full((1, 1, 128),
                           (first * b_scalar - c_scalar) / 128.0,
                           dtype=jnp.float32)


def _main_kernel(row_ref, col_ref, out_ref, acc_ref, *, jj_n, scale_a):
    i = pl.program_id(0)
    jj = pl.program_id(1)
    ni = pl.num_programs(0)

    @pl.when(jj == 0)
    def _init():
        acc_ref[...] = jnp.zeros_like(acc_ref)

    def _tile(lhs, c0):
        arg = jax.lax.dot_general(lhs, col_ref[c0:c0 + 512, :],
                                  (((1,), (1,)), ((), ())),
                                  preferred_element_type=jnp.float32)
        phi = jax.lax.rsqrt(jnp.maximum(arg.astype(jnp.bfloat16), 1.0))
        return (phi[:, 0:128] + phi[:, 128:256]) + (phi[:, 256:384]
                                                    + phi[:, 384:512])

    @pl.when(jj == 0)
    def _diag():
        rows = row_ref[...]
        nsub = rows.shape[0] // 512
        for mt in range(nsub):
            lhs = rows[mt * 512:(mt + 1) * 512, :]
            red = None
            for nt in range(mt, nsub):
                h = _tile(lhs, nt * 512)
                if nt > mt:
                    h = h + h
                red = h if red is None else red + h
            acc_ref[mt * 512:(mt + 1) * 512, :] += (
                scale_a * red.astype(jnp.float32))

    active = jnp.logical_and(
        jj > 0, jnp.logical_or(jj < jj_n - 1, i < ni // 2))

    @pl.when(active)
    def _offdiag():
        rows = row_ref[...]
        red = None
        for c0 in range(0, col_ref.shape[0], 512):
            h = _tile(rows, c0)
            red = h if red is None else red + h
        acc_ref[...] += (2.0 * scale_a) * red.astype(jnp.float32)

    @pl.when(jj == jj_n - 1)
    def _flush():
        out_ref[...] = jnp.sum(acc_ref[...], axis=0, keepdims=True
                               ).reshape(1, 1, 128)


@jax.jit
def kernel(X, means, variances, logit_probs):
    n, d = X.shape
    k = means.shape[0]
    gamma = float(np.power(4.0 / (3.0 * n / k), 0.4))
    c = 1.0 / ((2 * d - 3) * gamma)
    scale_a = 1.0 / (n * n * np.sqrt(TWO_PI * 2.0 * gamma))

    ti = 2048
    ni = n // ti
    jj_n = ni // 2 + 1

    tb = 2048
    nb = n // tb
    aug_row, aug_col, bc_parts = pl.pallas_call(
        functools.partial(_aug_kernel, c=c, gamma=gamma, n=n, d=d, k=k),
        grid=(nb,),
        in_specs=[pl.BlockSpec((tb, d), lambda b: (b, 0)),
                  pl.BlockSpec((k, d), lambda b: (0, 0)),
                  pl.BlockSpec((1, k), lambda b: (0, 0)),
                  pl.BlockSpec((1, k), lambda b: (0, 0))],
        out_specs=[pl.BlockSpec((tb, 68), lambda b: (b, 0)),
                   pl.BlockSpec((tb, 68), lambda b: (b, 0)),
                   pl.BlockSpec((1, 1, 128), lambda b: (b, 0, 0))],
        out_shape=[jax.ShapeDtypeStruct((n, 68), jnp.float8_e4m3fn),
                   jax.ShapeDtypeStruct((n, 68), jnp.float8_e4m3fn),
                   jax.ShapeDtypeStruct((nb, 1, 128), jnp.float32)],
        compiler_params=pltpu.CompilerParams(
            dimension_semantics=("arbitrary",)),
        name="cs_aug",
    )(X, means, variances.reshape(1, k), logit_probs.reshape(1, k))

    partials = pl.pallas_call(
        functools.partial(_main_kernel, jj_n=jj_n, scale_a=scale_a),
        grid=(ni, jj_n),
        in_specs=[
            pl.BlockSpec((ti, 68), lambda i, jj: (i, 0)),
            pl.BlockSpec((ti, 68), lambda i, jj: ((i + jj) % ni, 0)),
        ],
        out_specs=pl.BlockSpec((1, 1, 128), lambda i, jj: (i, 0, 0)),
        out_shape=jax.ShapeDtypeStruct((ni, 1, 128), jnp.float32),
        scratch_shapes=[pltpu.VMEM((ti, 128), jnp.float32)],
        compiler_params=pltpu.CompilerParams(
            dimension_semantics=("parallel", "arbitrary")),
        name="cs_pairwise",
    )(aug_row, aug_col)

    return jnp.log(jnp.sum(partials) + jnp.sum(bc_parts))
